# baseline (speedup 1.0000x reference)
import jax
import jax.numpy as jnp
from jax.experimental import pallas as pl
from jax.experimental.pallas import tpu as pltpu

EMB = 32
HID = 32
NUM_TAGS = 7
OUT_PAD = 8
NCHAIN = 1


def _gate_chunks(w, h):
    return (w[..., 0:h], w[..., h:2 * h], w[..., 2 * h:3 * h], w[..., 3 * h:4 * h])


def _bilstm_kernel(x_ref, h0_ref, c0_ref, wxa_ref, wxb_ref, b_ref, whh_ref,
                   wla_ref, wlb_ref, blin_ref, out_ref):
    SB, _ = x_ref.shape
    B = h0_ref.shape[0]
    H = whh_ref.shape[0] // 2
    S = SB // B
    BK = B // NCHAIN

    x = x_ref[...]
    gx = jnp.dot(x, jnp.concatenate([wxa_ref[...], wxb_ref[...]], axis=1),
                 preferred_element_type=jnp.float32)
    gxa = gx[:, 0:8 * H]
    gxb = gx[:, 8 * H:16 * H]
    bb = b_ref[...]
    whh = whh_ref[...]

    hks = [h0_ref[pl.ds(k * BK, BK), :] for k in range(NCHAIN)]
    cks = [c0_ref[pl.ds(k * BK, BK), :] for k in range(NCHAIN)]
    hs = [[] for _ in range(NCHAIN)]

    for t in range(S):
        rt = S - 1 - t
        for k in range(NCHAIN):
            h = hks[k]
            c = cks[k]
            gates = (gxa[t * B + k * BK:t * B + (k + 1) * BK]
                     + gxb[rt * B + k * BK:rt * B + (k + 1) * BK]
                     + bb
                     + jnp.dot(h, whh, preferred_element_type=jnp.float32))
            sig = jax.nn.sigmoid(gates[:, 0:6 * H])
            i = sig[:, 0:2 * H]
            f = sig[:, 2 * H:4 * H]
            o = sig[:, 4 * H:6 * H]
            g = jnp.tanh(gates[:, 6 * H:8 * H])
            c = f * c + i * g
            h = o * jnp.tanh(c)
            cks[k] = c
            hks[k] = h
            hs[k].append(h)

    wla = wla_ref[...]
    wlb = wlb_ref[...]
    bl = blin_ref[...]
    for k in range(NCHAIN):
        hs_fwd = jnp.concatenate(hs[k], axis=0)
        hs_bwd = jnp.concatenate(hs[k][::-1], axis=0)
        feats = (jnp.dot(hs_fwd, wla, preferred_element_type=jnp.float32)
                 + jnp.dot(hs_bwd, wlb, preferred_element_type=jnp.float32)
                 + bl)
        for t in range(S):
            out_ref[pl.ds(t * B + k * BK, BK), :] = feats[t * BK:(t + 1) * BK]


@jax.jit
def _run(sentence, word_emb, wih_f, whh_f, b_f, wih_b, whh_b, b_b,
         wlin, blin, h0, c0):
    B, S = sentence.shape
    E, H = EMB, HID

    Bh = B // 2
    st = sentence.T
    x1 = word_emb.at[st[:, :Bh].reshape(S * Bh)].get(
        mode="promise_in_bounds")
    x2 = word_emb.at[st[:, Bh:].reshape(S * Bh)].get(
        mode="promise_in_bounds")

    i_f, f_f, g_f, o_f = _gate_chunks(wih_f, H)
    i_b, f_b, g_b, o_b = _gate_chunks(wih_b, H)
    zE = jnp.zeros((E, H), jnp.float32)
    wxa = jnp.concatenate([i_f, zE, f_f, zE, o_f, zE, g_f, zE], axis=1)
    wxb = jnp.concatenate([zE, i_b, zE, f_b, zE, o_b, zE, g_b], axis=1)

    hi_f, hf_f, hg_f, ho_f = _gate_chunks(whh_f, H)
    hi_b, hf_b, hg_b, ho_b = _gate_chunks(whh_b, H)
    zH = jnp.zeros((H, H), jnp.float32)
    whh = jnp.concatenate([
        jnp.concatenate([hi_f, zH, hf_f, zH, ho_f, zH, hg_f, zH], axis=1),
        jnp.concatenate([zH, hi_b, zH, hf_b, zH, ho_b, zH, hg_b], axis=1)],
        axis=0)

    bi_f, bf_f, bg_f, bo_f = _gate_chunks(b_f, H)
    bi_b, bf_b, bg_b, bo_b = _gate_chunks(b_b, H)
    b = jnp.concatenate([bi_f, bi_b, bf_f, bf_b, bo_f, bo_b, bg_f, bg_b],
                        axis=1)

    wpad = jnp.pad(wlin, ((0, 0), (0, OUT_PAD - NUM_TAGS)))
    zHT = jnp.zeros((H, OUT_PAD), jnp.float32)
    wla = jnp.concatenate([wpad[0:H], zHT], axis=0)
    wlb = jnp.concatenate([zHT, wpad[H:2 * H]], axis=0)
    blin_p = jnp.pad(blin, ((0, 0), (0, OUT_PAD - NUM_TAGS)))

    h0_cat = jnp.concatenate([h0[0], h0[1]], axis=1)
    c0_cat = jnp.concatenate([c0[0], c0[1]], axis=1)

    def full(shape):
        nd = len(shape)
        return pl.BlockSpec(shape, lambda i, nd=nd: (0,) * nd)

    def half(xh, h0h, c0h):
        inputs = (xh, h0h, c0h, wxa, wxb, b, whh, wla, wlb, blin_p)
        return pl.pallas_call(
            _bilstm_kernel,
            out_shape=jax.ShapeDtypeStruct((S * Bh, OUT_PAD), jnp.float32),
            grid=(1,),
            in_specs=[full(v.shape) for v in inputs],
            out_specs=full((S * Bh, OUT_PAD)),
            compiler_params=pltpu.CompilerParams(
                dimension_semantics=("arbitrary",)),
        )(*inputs)

    f1 = half(x1, h0_cat[:Bh], c0_cat[:Bh]).reshape(S, Bh, OUT_PAD)
    f2 = half(x2, h0_cat[Bh:], c0_cat[Bh:]).reshape(S, Bh, OUT_PAD)

    feats = jnp.concatenate([f1, f2], axis=1)
    return jnp.transpose(feats, (1, 0, 2))[:, :, :NUM_TAGS]


def kernel(sentence, word_emb, wih_f, whh_f, b_f, wih_b, whh_b, b_b,
           wlin, blin, h0, c0):
    return _run(sentence, word_emb, wih_f, whh_f, b_f, wih_b, whh_b, b_b,
                wlin, blin, h0, c0)


# speedup vs baseline: 1.2881x; 1.2881x over previous
import jax
import jax.numpy as jnp
from jax.experimental import pallas as pl
from jax.experimental.pallas import tpu as pltpu

EMB = 32
HID = 32
NUM_TAGS = 7
OUT_PAD = 8
NCHAIN = 2


def _gate_chunks(w, h):
    return (w[..., 0:h], w[..., h:2 * h], w[..., 2 * h:3 * h], w[..., 3 * h:4 * h])


def _bilstm_kernel(x_ref, h0_ref, c0_ref, wxa_ref, wxb_ref, b_ref, whh_ref,
                   wla_ref, wlb_ref, blin_ref, out_ref):
    SB, _ = x_ref.shape
    B = h0_ref.shape[0]
    H = whh_ref.shape[0] // 2
    S = SB // B
    BK = B // NCHAIN

    x = x_ref[...]
    wxa = wxa_ref[...]
    wxb = wxb_ref[...]
    bb = b_ref[...]
    whh = whh_ref[...]

    hks = [h0_ref[pl.ds(k * BK, BK), :] for k in range(NCHAIN)]
    cks = [c0_ref[pl.ds(k * BK, BK), :] for k in range(NCHAIN)]
    hs = [[] for _ in range(NCHAIN)]

    for t in range(S):
        rt = S - 1 - t
        for k in range(NCHAIN):
            h = hks[k]
            c = cks[k]
            gates = (jnp.dot(x[t * B + k * BK:t * B + (k + 1) * BK],
                             wxa, preferred_element_type=jnp.float32)
                     + jnp.dot(x[rt * B + k * BK:rt * B + (k + 1) * BK],
                               wxb, preferred_element_type=jnp.float32)
                     + bb
                     + jnp.dot(h, whh, preferred_element_type=jnp.float32))
            sig = jax.nn.sigmoid(gates[:, 0:6 * H])
            i = sig[:, 0:2 * H]
            f = sig[:, 2 * H:4 * H]
            o = sig[:, 4 * H:6 * H]
            g = jnp.tanh(gates[:, 6 * H:8 * H])
            c = f * c + i * g
            h = o * jnp.tanh(c)
            cks[k] = c
            hks[k] = h
            hs[k].append(h)

    wla = wla_ref[...]
    wlb = wlb_ref[...]
    bl = blin_ref[...]
    for k in range(NCHAIN):
        hs_fwd = jnp.concatenate(hs[k], axis=0)
        hs_bwd = jnp.concatenate(hs[k][::-1], axis=0)
        feats = (jnp.dot(hs_fwd, wla, preferred_element_type=jnp.float32)
                 + jnp.dot(hs_bwd, wlb, preferred_element_type=jnp.float32)
                 + bl)
        for t in range(S):
            out_ref[pl.ds(t * B + k * BK, BK), :] = feats[t * BK:(t + 1) * BK]


@jax.jit
def _run(sentence, word_emb, wih_f, whh_f, b_f, wih_b, whh_b, b_b,
         wlin, blin, h0, c0):
    B, S = sentence.shape
    E, H = EMB, HID

    x_tm = word_emb.at[sentence.T.reshape(S * B)].get(
        mode="promise_in_bounds")

    i_f, f_f, g_f, o_f = _gate_chunks(wih_f, H)
    i_b, f_b, g_b, o_b = _gate_chunks(wih_b, H)
    zE = jnp.zeros((E, H), jnp.float32)
    wxa = jnp.concatenate([i_f, zE, f_f, zE, o_f, zE, g_f, zE], axis=1)
    wxb = jnp.concatenate([zE, i_b, zE, f_b, zE, o_b, zE, g_b], axis=1)

    hi_f, hf_f, hg_f, ho_f = _gate_chunks(whh_f, H)
    hi_b, hf_b, hg_b, ho_b = _gate_chunks(whh_b, H)
    zH = jnp.zeros((H, H), jnp.float32)
    whh = jnp.concatenate([
        jnp.concatenate([hi_f, zH, hf_f, zH, ho_f, zH, hg_f, zH], axis=1),
        jnp.concatenate([zH, hi_b, zH, hf_b, zH, ho_b, zH, hg_b], axis=1)],
        axis=0)

    bi_f, bf_f, bg_f, bo_f = _gate_chunks(b_f, H)
    bi_b, bf_b, bg_b, bo_b = _gate_chunks(b_b, H)
    b = jnp.concatenate([bi_f, bi_b, bf_f, bf_b, bo_f, bo_b, bg_f, bg_b],
                        axis=1)

    wpad = jnp.pad(wlin, ((0, 0), (0, OUT_PAD - NUM_TAGS)))
    zHT = jnp.zeros((H, OUT_PAD), jnp.float32)
    wla = jnp.concatenate([wpad[0:H], zHT], axis=0)
    wlb = jnp.concatenate([zHT, wpad[H:2 * H]], axis=0)
    blin_p = jnp.pad(blin, ((0, 0), (0, OUT_PAD - NUM_TAGS)))

    h0_cat = jnp.concatenate([h0[0], h0[1]], axis=1)
    c0_cat = jnp.concatenate([c0[0], c0[1]], axis=1)

    inputs = (x_tm, h0_cat, c0_cat, wxa, wxb, b, whh, wla, wlb, blin_p)

    def full(shape):
        nd = len(shape)
        return pl.BlockSpec(shape, lambda i, nd=nd: (0,) * nd)

    feats_tm = pl.pallas_call(
        _bilstm_kernel,
        out_shape=jax.ShapeDtypeStruct((S * B, OUT_PAD), jnp.float32),
        grid=(1,),
        in_specs=[full(v.shape) for v in inputs],
        out_specs=full((S * B, OUT_PAD)),
        compiler_params=pltpu.CompilerParams(
            dimension_semantics=("arbitrary",)),
    )(*inputs)

    feats = feats_tm.reshape(S, B, OUT_PAD)
    return jnp.transpose(feats, (1, 0, 2))[:, :, :NUM_TAGS]


def kernel(sentence, word_emb, wih_f, whh_f, b_f, wih_b, whh_b, b_b,
           wlin, blin, h0, c0):
    return _run(sentence, word_emb, wih_f, whh_f, b_f, wih_b, whh_b, b_b,
                wlin, blin, h0, c0)


# speedup vs baseline: 1.3004x; 1.0095x over previous
import jax
import jax.numpy as jnp
from jax.experimental import pallas as pl
from jax.experimental.pallas import tpu as pltpu

EMB = 32
HID = 32
NUM_TAGS = 7
OUT_PAD = 8
NCHAIN = 2


def _gate_chunks(w, h):
    return (w[..., 0:h], w[..., h:2 * h], w[..., 2 * h:3 * h], w[..., 3 * h:4 * h])


def _bilstm_kernel(x_ref, h0_ref, c0_ref, wxa_ref, wxb_ref, b_ref, whh_ref,
                   wla_ref, wlb_ref, blin_ref, out_ref):
    SB, _ = x_ref.shape
    B = h0_ref.shape[0]
    H = whh_ref.shape[0] // 2
    S = SB // B
    BK = B // NCHAIN

    x = x_ref[...]
    wxa = wxa_ref[...]
    wxb = wxb_ref[...]
    bb = b_ref[...]
    whh = whh_ref[...]

    hks = [h0_ref[pl.ds(k * BK, BK), :] for k in range(NCHAIN)]
    cks = [c0_ref[pl.ds(k * BK, BK), :] for k in range(NCHAIN)]
    hs = [[] for _ in range(NCHAIN)]

    for t in range(S):
        rt = S - 1 - t
        gx_t = (jnp.dot(x[t * B:(t + 1) * B], wxa,
                        preferred_element_type=jnp.float32)
                + jnp.dot(x[rt * B:(rt + 1) * B], wxb,
                          preferred_element_type=jnp.float32)
                + bb)
        for k in range(NCHAIN):
            h = hks[k]
            c = cks[k]
            gates = (gx_t[k * BK:(k + 1) * BK]
                     + jnp.dot(h, whh, preferred_element_type=jnp.float32))
            sig = jax.nn.sigmoid(gates[:, 0:6 * H])
            i = sig[:, 0:2 * H]
            f = sig[:, 2 * H:4 * H]
            o = sig[:, 4 * H:6 * H]
            g = jnp.tanh(gates[:, 6 * H:8 * H])
            c = f * c + i * g
            h = o * jnp.tanh(c)
            cks[k] = c
            hks[k] = h
            hs[k].append(h)

    wla = wla_ref[...]
    wlb = wlb_ref[...]
    bl = blin_ref[...]
    for k in range(NCHAIN):
        hs_fwd = jnp.concatenate(hs[k], axis=0)
        hs_bwd = jnp.concatenate(hs[k][::-1], axis=0)
        feats = (jnp.dot(hs_fwd, wla, preferred_element_type=jnp.float32)
                 + jnp.dot(hs_bwd, wlb, preferred_element_type=jnp.float32)
                 + bl)
        for t in range(S):
            out_ref[pl.ds(t * B + k * BK, BK), :] = feats[t * BK:(t + 1) * BK]


@jax.jit
def _run(sentence, word_emb, wih_f, whh_f, b_f, wih_b, whh_b, b_b,
         wlin, blin, h0, c0):
    B, S = sentence.shape
    E, H = EMB, HID

    x_tm = word_emb.at[sentence.T.reshape(S * B)].get(
        mode="promise_in_bounds")

    i_f, f_f, g_f, o_f = _gate_chunks(wih_f, H)
    i_b, f_b, g_b, o_b = _gate_chunks(wih_b, H)
    zE = jnp.zeros((E, H), jnp.float32)
    wxa = jnp.concatenate([i_f, zE, f_f, zE, o_f, zE, g_f, zE], axis=1)
    wxb = jnp.concatenate([zE, i_b, zE, f_b, zE, o_b, zE, g_b], axis=1)

    hi_f, hf_f, hg_f, ho_f = _gate_chunks(whh_f, H)
    hi_b, hf_b, hg_b, ho_b = _gate_chunks(whh_b, H)
    zH = jnp.zeros((H, H), jnp.float32)
    whh = jnp.concatenate([
        jnp.concatenate([hi_f, zH, hf_f, zH, ho_f, zH, hg_f, zH], axis=1),
        jnp.concatenate([zH, hi_b, zH, hf_b, zH, ho_b, zH, hg_b], axis=1)],
        axis=0)

    bi_f, bf_f, bg_f, bo_f = _gate_chunks(b_f, H)
    bi_b, bf_b, bg_b, bo_b = _gate_chunks(b_b, H)
    b = jnp.concatenate([bi_f, bi_b, bf_f, bf_b, bo_f, bo_b, bg_f, bg_b],
                        axis=1)

    wpad = jnp.pad(wlin, ((0, 0), (0, OUT_PAD - NUM_TAGS)))
    zHT = jnp.zeros((H, OUT_PAD), jnp.float32)
    wla = jnp.concatenate([wpad[0:H], zHT], axis=0)
    wlb = jnp.concatenate([zHT, wpad[H:2 * H]], axis=0)
    blin_p = jnp.pad(blin, ((0, 0), (0, OUT_PAD - NUM_TAGS)))

    h0_cat = jnp.concatenate([h0[0], h0[1]], axis=1)
    c0_cat = jnp.concatenate([c0[0], c0[1]], axis=1)

    inputs = (x_tm, h0_cat, c0_cat, wxa, wxb, b, whh, wla, wlb, blin_p)

    def full(shape):
        nd = len(shape)
        return pl.BlockSpec(shape, lambda i, nd=nd: (0,) * nd)

    feats_tm = pl.pallas_call(
        _bilstm_kernel,
        out_shape=jax.ShapeDtypeStruct((S * B, OUT_PAD), jnp.float32),
        grid=(1,),
        in_specs=[full(v.shape) for v in inputs],
        out_specs=full((S * B, OUT_PAD)),
        compiler_params=pltpu.CompilerParams(
            dimension_semantics=("arbitrary",)),
    )(*inputs)

    feats = feats_tm.reshape(S, B, OUT_PAD)
    return jnp.transpose(feats, (1, 0, 2))[:, :, :NUM_TAGS]


def kernel(sentence, word_emb, wih_f, whh_f, b_f, wih_b, whh_b, b_b,
           wlin, blin, h0, c0):
    return _run(sentence, word_emb, wih_f, whh_f, b_f, wih_b, whh_b, b_b,
                wlin, blin, h0, c0)
